# baseline (speedup 1.0000x reference)
import functools

import jax
import jax.numpy as jnp
from jax import lax
from jax.experimental import pallas as pl
from jax.experimental.pallas import tpu as pltpu
from jax.experimental.pallas import tpu_sc as plsc

N = 2048
D = 256
C = 3
CH_STRIDE = D * D * D

NC, NS, L = 2, 16, 16
NW = NC * NS
PER_W = N // NW
VECS = PER_W // L

_mesh = plsc.VectorSubcoreMesh(core_axis_name="c", subcore_axis_name="s",
                               num_cores=NC, num_subcores=NS)


@functools.partial(
    pl.kernel,
    mesh=_mesh,
    compiler_params=pltpu.CompilerParams(needs_layout_passes=False),
    out_type=jax.ShapeDtypeStruct((C, N), jnp.float32),
    scratch_types=[
        pltpu.VMEM((PER_W,), jnp.int32),
        pltpu.VMEM((PER_W,), jnp.int32),
        pltpu.VMEM((PER_W,), jnp.int32),
        pltpu.VMEM((PER_W,), jnp.int32),
        pltpu.VMEM((PER_W,), jnp.int32),
        pltpu.VMEM((PER_W,), jnp.int32),
        pltpu.VMEM((L,), jnp.float32),
        pltpu.VMEM((PER_W,), jnp.int32),
        pltpu.VMEM((PER_W,), jnp.int32),
        pltpu.VMEM((PER_W,), jnp.int32),
        pltpu.VMEM((PER_W,), jnp.float32),
        pltpu.VMEM((PER_W,), jnp.float32),
        pltpu.VMEM((PER_W,), jnp.float32),
        pltpu.VMEM((C, PER_W), jnp.float32),
        pltpu.SemaphoreType.DMA,
        pltpu.SemaphoreType.DMA,
        pltpu.SemaphoreType.DMA,
    ],
)
def _trl_kernel(mlT_hbm, flT_hbm, field_hbm, spac_hbm, out_hbm,
                xm_v, ym_v, zm_v, xf_v, yf_v, zf_v, ms_v,
                i0_v, i1_v, i2_v, d0_v, d1_v, d2_v, ob_v,
                sem_g, sem_m, sem_f):
    wid = lax.axis_index("s") * NC + lax.axis_index("c")
    n0 = wid * PER_W

    cpm = [pltpu.async_copy(mlT_hbm.at[c, pl.ds(n0, PER_W)], dst, sem_m)
           for c, dst in enumerate((xm_v, ym_v, zm_v))]
    cpf = [pltpu.async_copy(flT_hbm.at[c, pl.ds(n0, PER_W)], dst, sem_f)
           for c, dst in enumerate((xf_v, yf_v, zf_v))]
    cps = pltpu.async_copy(spac_hbm, ms_v.at[pl.ds(0, C)], sem_f)

    for cp in cpm:
        cp.wait()
    for j in range(VECS):
        sl = pl.ds(j * L, L)
        x, y, z = xm_v[sl], ym_v[sl], zm_v[sl]
        tile = ((y >> 3) * 2 + (z >> 7)) * 1024
        lin = x * (D * D) + tile + (y & 7) * 128 + (z & 127)
        i0_v[sl] = lin
        i1_v[sl] = lin + CH_STRIDE
        i2_v[sl] = lin + 2 * CH_STRIDE

    gathers = [pltpu.async_copy(field_hbm.at[iv], dv, sem_g)
               for iv, dv in ((i0_v, d0_v), (i1_v, d1_v), (i2_v, d2_v))]

    cps.wait()
    spv = ms_v[...]
    for cp in cpf:
        cp.wait()

    outs = []
    for c, (cp, mv, fv, dv) in enumerate(
            zip(gathers, (xm_v, ym_v, zm_v), (xf_v, yf_v, zf_v),
                (d0_v, d1_v, d2_v))):
        cp.wait()
        for j in range(VECS):
            sl = pl.ds(j * L, L)
            diff = (mv[sl] - fv[sl]).astype(jnp.float32)
            ob_v[c, sl] = (diff + dv[sl]) * spv[c]
        outs.append(pltpu.async_copy(ob_v.at[c],
                                     out_hbm.at[c, pl.ds(n0, PER_W)], sem_m))
    for cp in outs:
        cp.wait()


def kernel(fixed_landmarks, moving_landmarks, displacement_field,
           fixed_spacing, moving_spacing):
    del fixed_spacing
    field_flat = (displacement_field
                  .reshape(C, D, D // 8, 8, D // 128, 128)
                  .transpose(0, 1, 2, 4, 3, 5)
                  .reshape(-1))
    outT = _trl_kernel(moving_landmarks.T, fixed_landmarks.T, field_flat,
                       moving_spacing.astype(jnp.float32))
    return outT.T


# speedup vs baseline: 1.0030x; 1.0030x over previous
import functools

import jax
import jax.numpy as jnp
from jax import lax
from jax.experimental import pallas as pl
from jax.experimental.pallas import tpu as pltpu
from jax.experimental.pallas import tpu_sc as plsc

N = 2048
D = 256
C = 3
CH_STRIDE = D * D * D

NC, NS, L = 2, 16, 16
NW = NC * NS
PER_W = N // NW
VECS = PER_W // L

_mesh = plsc.VectorSubcoreMesh(core_axis_name="c", subcore_axis_name="s",
                               num_cores=NC, num_subcores=NS)


@functools.partial(
    pl.kernel,
    mesh=_mesh,
    compiler_params=pltpu.CompilerParams(needs_layout_passes=False),
    out_type=jax.ShapeDtypeStruct((C, N), jnp.float32),
    scratch_types=[
        pltpu.VMEM((PER_W,), jnp.int32),
        pltpu.VMEM((PER_W,), jnp.int32),
        pltpu.VMEM((PER_W,), jnp.int32),
        pltpu.VMEM((PER_W,), jnp.int32),
        pltpu.VMEM((PER_W,), jnp.int32),
        pltpu.VMEM((PER_W,), jnp.int32),
        pltpu.VMEM((L,), jnp.float32),
        pltpu.VMEM((PER_W,), jnp.int32),
        pltpu.VMEM((PER_W,), jnp.int32),
        pltpu.VMEM((PER_W,), jnp.int32),
        pltpu.VMEM((PER_W,), jnp.float32),
        pltpu.VMEM((PER_W,), jnp.float32),
        pltpu.VMEM((PER_W,), jnp.float32),
        pltpu.VMEM((C, PER_W), jnp.float32),
        pltpu.SemaphoreType.DMA,
        pltpu.SemaphoreType.DMA,
        pltpu.SemaphoreType.DMA,
    ],
)
def _trl_kernel(mlT_hbm, flT_hbm, field_hbm, spac_hbm, out_hbm,
                xm_v, ym_v, zm_v, xf_v, yf_v, zf_v, ms_v,
                i0_v, i1_v, i2_v, d0_v, d1_v, d2_v, ob_v,
                sem_g, sem_m, sem_f):
    wid = lax.axis_index("s") * NC + lax.axis_index("c")
    n0 = wid * PER_W

    cpm = [pltpu.async_copy(mlT_hbm.at[c, pl.ds(n0, PER_W)], dst, sem_m)
           for c, dst in enumerate((xm_v, ym_v, zm_v))]
    cpf = [pltpu.async_copy(flT_hbm.at[c, pl.ds(n0, PER_W)], dst, sem_f)
           for c, dst in enumerate((xf_v, yf_v, zf_v))]
    cps = pltpu.async_copy(spac_hbm, ms_v.at[pl.ds(0, C)], sem_f)

    for cp in cpm:
        cp.wait()
    for j in range(VECS):
        sl = pl.ds(j * L, L)
        x, y, z = xm_v[sl], ym_v[sl], zm_v[sl]
        tile = ((y >> 3) * 2 + (z >> 7)) * 1024
        lin = x * (D * D) + tile + (y & 7) * 128 + (z & 127)
        i0_v[sl] = lin
        i1_v[sl] = lin + CH_STRIDE
        i2_v[sl] = lin + 2 * CH_STRIDE

    gathers = [pltpu.async_copy(field_hbm.at[iv], dv, sem_g)
               for iv, dv in ((i0_v, d0_v), (i1_v, d1_v), (i2_v, d2_v))]

    cps.wait()
    spv = ms_v[...]
    for cp in cpf:
        cp.wait()

    for cp in gathers:
        cp.wait()

    outs = []
    for c, (mv, fv, dv) in enumerate(
            zip((xm_v, ym_v, zm_v), (xf_v, yf_v, zf_v),
                (d0_v, d1_v, d2_v))):
        for j in range(VECS):
            sl = pl.ds(j * L, L)
            diff = (mv[sl] - fv[sl]).astype(jnp.float32)
            ob_v[c, sl] = (diff + dv[sl]) * spv[c]
        outs.append(pltpu.async_copy(ob_v.at[c],
                                     out_hbm.at[c, pl.ds(n0, PER_W)], sem_m))
    for cp in outs:
        cp.wait()


def kernel(fixed_landmarks, moving_landmarks, displacement_field,
           fixed_spacing, moving_spacing):
    del fixed_spacing
    field_flat = (displacement_field
                  .reshape(C, D, D // 8, 8, D // 128, 128)
                  .transpose(0, 1, 2, 4, 3, 5)
                  .reshape(-1))
    outT = _trl_kernel(moving_landmarks.T, fixed_landmarks.T, field_flat,
                       moving_spacing.astype(jnp.float32))
    return outT.T
